# Initial kernel scaffold; baseline (speedup 1.0000x reference)
import jax
import jax.numpy as jnp
from jax.experimental import pallas as pl


def kernel(item_entities, entiEmbs, att):
    raise NotImplementedError("write your pallas kernel here")


# speedup vs baseline: 9.0164x; 9.0164x over previous
import functools

import jax
import jax.numpy as jnp
from jax import lax
from jax.experimental import pallas as pl
from jax.experimental.pallas import tpu as pltpu
from jax.experimental.pallas import tpu_sc as plsc

N = 16384
D = 128
P = 32
NW = 32
IPW = N // NW
HALF = IPW // 2
NCH = HALF * P // 128
NG = HALF // 16


def _sc_body(ie_hbm, at_hbm, emb_hbm, out_hbm,
             idx_v, aw_v, buf0, buf1, out_v, sem0, sem1):
    cid = lax.axis_index("c")
    sid = lax.axis_index("s")
    wid = sid * 2 + cid

    def half_body(h, carry):
        slab = wid * 2 + h
        ibase = wid * IPW + h * HALF

        pltpu.sync_copy(ie_hbm.at[slab], idx_v)
        pltpu.sync_copy(at_hbm.at[slab], aw_v)
        pltpu.sync_copy(emb_hbm.at[pl.ds(ibase, HALF)], out_v)

        def start(c, buf, sem):
            pltpu.make_async_copy(emb_hbm.at[idx_v.at[c]], buf, sem).start()

        def wait(buf, sem):
            pltpu.make_async_copy(emb_hbm.at[idx_v.at[0]], buf, sem).wait()

        def compute(c, buf):
            def item_body(sub, c2):
                r = c * 4 + sub
                rb = sub * P
                w0 = aw_v[pl.ds(r * P, 16)]
                w1 = aw_v[pl.ds(r * P + 16, 16)]
                acc = [out_v[r, pl.ds(j * 16, 16)] for j in range(8)]
                for p in range(P):
                    wgt = w0[p] if p < 16 else w1[p - 16]
                    for j in range(8):
                        acc[j] = acc[j] + wgt * buf[rb + p, pl.ds(j * 16, 16)]
                for j in range(8):
                    out_v[r, pl.ds(j * 16, 16)] = acc[j]
                return c2
            lax.fori_loop(0, 4, item_body, 0)

        start(0, buf0, sem0)

        def c2_body(c2, carry2):
            c0 = c2 * 2
            start(c0 + 1, buf1, sem1)
            wait(buf0, sem0)
            compute(c0, buf0)

            @pl.when(c0 + 2 < NCH)
            def _():
                start(c0 + 2, buf0, sem0)
            wait(buf1, sem1)
            compute(c0 + 1, buf1)
            return carry2
        lax.fori_loop(0, NCH // 2, c2_body, 0)

        pltpu.sync_copy(out_v, out_hbm.at[pl.ds(ibase, HALF)])
        return carry
    lax.fori_loop(0, 2, half_body, 0)


def _softmax_body(x_ref, o_ref):
    x = x_ref[...]
    m = jnp.max(x, axis=1, keepdims=True)
    e = jnp.exp(x - m)
    o_ref[...] = e / jnp.sum(e, axis=1, keepdims=True)


def _softmax_tc(att):
    return pl.pallas_call(
        _softmax_body,
        out_shape=jax.ShapeDtypeStruct((N, P), jnp.float32),
        grid=(8,),
        in_specs=[pl.BlockSpec((N // 8, P), lambda i: (i, 0))],
        out_specs=pl.BlockSpec((N // 8, P), lambda i: (i, 0)),
    )(att)


@jax.jit
def kernel(item_entities, entiEmbs, att):
    ie = item_entities.reshape(NW * 2, NCH, 128)
    at = _softmax_tc(att).reshape(NW * 2, HALF * P)

    mesh = plsc.VectorSubcoreMesh(core_axis_name="c", subcore_axis_name="s")
    f = pl.kernel(
        _sc_body,
        out_type=jax.ShapeDtypeStruct((N, D), jnp.float32),
        mesh=mesh,
        scratch_types=[
            pltpu.VMEM((NCH, 128), jnp.int32),
            pltpu.VMEM((HALF * P,), jnp.float32),
            pltpu.VMEM((128, D), jnp.float32),
            pltpu.VMEM((128, D), jnp.float32),
            pltpu.VMEM((HALF, D), jnp.float32),
            pltpu.SemaphoreType.DMA,
            pltpu.SemaphoreType.DMA,
        ],
    )
    return f(ie, at, entiEmbs)
